# Initial kernel scaffold; baseline (speedup 1.0000x reference)
import jax
import jax.numpy as jnp
from jax.experimental import pallas as pl


def kernel(x, table, W, b):
    raise NotImplementedError("write your pallas kernel here")


# speedup vs baseline: 1.8879x; 1.8879x over previous
import functools

import jax
import jax.numpy as jnp
from jax import lax
from jax.experimental import pallas as pl
from jax.experimental.pallas import tpu as pltpu
from jax.experimental.pallas import tpu_sc as plsc

VOCAB = 1000000
EMB = 32
NCLASS = 100
BATCH = 4096
HIST = 200

NC = 2
NS = 16
NW = NC * NS
ROWS_PW = BATCH // NW

CHUNK0 = 104
CHUNK1 = 96


def _pool_kernel(x_hbm, table_hbm, out_hbm, idx_v, rows_v, pool_v, sem):
    wid = lax.axis_index("s") * NC + lax.axis_index("c")
    base = wid * ROWS_PW
    pltpu.sync_copy(x_hbm.at[pl.ds(base * HIST, ROWS_PW * HIST)], idx_v)

    inv = jnp.full((16,), 1.0 / HIST, dtype=jnp.float32)

    def row_body(r, carry):
        cp0 = pltpu.async_copy(
            table_hbm.at[idx_v.at[pl.ds(r * HIST, CHUNK0)]],
            rows_v.at[pl.ds(0, CHUNK0)], sem)
        cp1 = pltpu.async_copy(
            table_hbm.at[idx_v.at[pl.ds(r * HIST + CHUNK0, CHUNK1)]],
            rows_v.at[pl.ds(CHUNK0, CHUNK1)], sem)
        cp0.wait()
        cp1.wait()

        def acc_body(i, acc):
            a0, a1 = acc
            a0 = a0 + rows_v[i, pl.ds(0, 16)]
            a1 = a1 + rows_v[i, pl.ds(16, 16)]
            return (a0, a1)

        zero = jnp.zeros((16,), dtype=jnp.float32)
        a0, a1 = lax.fori_loop(0, HIST, acc_body, (zero, zero))
        pool_v[pl.ds(r * EMB, 16)] = a0 * inv
        pool_v[pl.ds(r * EMB + 16, 16)] = a1 * inv
        return carry

    lax.fori_loop(0, ROWS_PW, row_body, 0)
    pltpu.sync_copy(pool_v, out_hbm.at[pl.ds(base * EMB, ROWS_PW * EMB)])


@jax.jit
def _pool(x, table):
    mesh = plsc.VectorSubcoreMesh(core_axis_name="c", subcore_axis_name="s")
    return pl.kernel(
        _pool_kernel,
        mesh=mesh,
        compiler_params=pltpu.CompilerParams(use_tc_tiling_on_sc=False),
        out_type=jax.ShapeDtypeStruct((BATCH * EMB,), jnp.float32),
        scratch_types=[
            pltpu.VMEM((ROWS_PW * HIST,), jnp.int32),
            pltpu.VMEM((HIST, EMB), jnp.float32),
            pltpu.VMEM((ROWS_PW * EMB,), jnp.float32),
            pltpu.SemaphoreType.DMA,
        ],
    )(x, table)


def _mm_kernel(p_ref, w_ref, b_ref, o_ref):
    o_ref[...] = lax.dot_general(
        p_ref[...], w_ref[...],
        (((1,), (1,)), ((), ())),
        preferred_element_type=jnp.float32,
    ) + b_ref[...]


@jax.jit
def _linear(pooled, W, b):
    return pl.pallas_call(
        _mm_kernel,
        out_shape=jax.ShapeDtypeStruct((BATCH, NCLASS), jnp.float32),
    )(pooled, W, b.reshape(1, NCLASS))


def kernel(x, table, W, b):
    pooled = _pool(x.reshape(BATCH * HIST), table).reshape(BATCH, EMB)
    return _linear(pooled, W, b)


# speedup vs baseline: 2.6975x; 1.4289x over previous
import functools

import jax
import jax.numpy as jnp
from jax import lax
from jax.experimental import pallas as pl
from jax.experimental.pallas import tpu as pltpu
from jax.experimental.pallas import tpu_sc as plsc

VOCAB = 1000000
EMB = 32
NCLASS = 100
BATCH = 4096
HIST = 200

TBW = 2048
TNB = 123
QP = TBW * TNB
VPAD = 4 * QP

NC = 2
NS = 16
NW = NC * NS
ROWS_PW = BATCH // NW
IDX_PW = ROWS_PW * HIST

CHUNK0 = 104
CHUNK1 = 96


def _transpose_kernel(i0, i1, i2, i3, o_ref):
    for q, iq in enumerate((i0, i1, i2, i3)):
        o_ref[:, 32 * q:32 * (q + 1)] = iq[...].T


LASTB = (VOCAB - 1) // TBW


@jax.jit
def _transpose(tT):
    return pl.pallas_call(
        _transpose_kernel,
        grid=(TNB,),
        in_specs=[pl.BlockSpec(
            (32, TBW),
            (lambda j, q=q: (0, jnp.minimum(TNB * q + j, LASTB))))
            for q in range(4)],
        out_specs=pl.BlockSpec((TBW, 128), lambda j: (j, 0)),
        out_shape=jax.ShapeDtypeStruct((QP, 128), jnp.float32),
    )(tT, tT, tT, tT)


def _pool_kernel(x_hbm, table_hbm, out_hbm, idx_v, rows_v, pool_v, sem):
    wid = lax.axis_index("s") * NC + lax.axis_index("c")
    base = wid * ROWS_PW
    pltpu.sync_copy(x_hbm.at[pl.ds(base * HIST, IDX_PW)], idx_v)

    def remap_body(i, carry):
        v = idx_v[pl.ds(i * 16, 16)]
        w = v * 4
        for k in (1, 2, 3):
            w = w - jnp.where(v >= k * QP, jnp.int32(4 * QP - 1), jnp.int32(0))
        idx_v[pl.ds(i * 16, 16)] = w
        return carry

    lax.fori_loop(0, IDX_PW // 16, remap_body, 0)

    inv = jnp.full((16,), 1.0 / HIST, dtype=jnp.float32)

    def row_body(r, carry):
        cp0 = pltpu.async_copy(
            table_hbm.at[idx_v.at[pl.ds(r * HIST, CHUNK0)]],
            rows_v.at[pl.ds(0, CHUNK0)], sem)
        cp1 = pltpu.async_copy(
            table_hbm.at[idx_v.at[pl.ds(r * HIST + CHUNK0, CHUNK1)]],
            rows_v.at[pl.ds(CHUNK0, CHUNK1)], sem)
        cp0.wait()
        cp1.wait()

        def acc_body(i, acc):
            a0, a1 = acc
            a0 = a0 + rows_v[i, pl.ds(0, 16)]
            a1 = a1 + rows_v[i, pl.ds(16, 16)]
            return (a0, a1)

        zero = jnp.zeros((16,), dtype=jnp.float32)
        a0, a1 = lax.fori_loop(0, HIST, acc_body, (zero, zero))
        pool_v[pl.ds(r * EMB, 16)] = a0 * inv
        pool_v[pl.ds(r * EMB + 16, 16)] = a1 * inv
        return carry

    lax.fori_loop(0, ROWS_PW, row_body, 0)
    pltpu.sync_copy(pool_v, out_hbm.at[pl.ds(base * EMB, ROWS_PW * EMB)])


@jax.jit
def _pool(x, table):
    mesh = plsc.VectorSubcoreMesh(core_axis_name="c", subcore_axis_name="s")
    return pl.kernel(
        _pool_kernel,
        mesh=mesh,
        compiler_params=pltpu.CompilerParams(use_tc_tiling_on_sc=False),
        out_type=jax.ShapeDtypeStruct((BATCH * EMB,), jnp.float32),
        scratch_types=[
            pltpu.VMEM((IDX_PW,), jnp.int32),
            pltpu.VMEM((HIST, EMB), jnp.float32),
            pltpu.VMEM((ROWS_PW * EMB,), jnp.float32),
            pltpu.SemaphoreType.DMA,
        ],
    )(x, table)


def _mm_kernel(p_ref, w_ref, b_ref, o_ref):
    o_ref[...] = lax.dot_general(
        p_ref[...], w_ref[...],
        (((1,), (1,)), ((), ())),
        preferred_element_type=jnp.float32,
    ) + b_ref[...]


@jax.jit
def _linear(pooled, W, b):
    return pl.pallas_call(
        _mm_kernel,
        out_shape=jax.ShapeDtypeStruct((BATCH, NCLASS), jnp.float32),
    )(pooled, W, b.reshape(1, NCLASS))


def kernel(x, table, W, b):
    z = _transpose(table.T)
    pooled = _pool(x.reshape(BATCH * HIST),
                   z.reshape(VPAD, EMB)).reshape(BATCH, EMB)
    return _linear(pooled, W, b)


# speedup vs baseline: 3.8001x; 1.4087x over previous
import functools

import jax
import jax.numpy as jnp
from jax import lax
from jax.experimental import pallas as pl
from jax.experimental.pallas import tpu as pltpu
from jax.experimental.pallas import tpu_sc as plsc

VOCAB = 1000000
EMB = 32
NCLASS = 100
BATCH = 4096
HIST = 200

TBW = 2048
TNB = 123
QP = TBW * TNB
VPAD = 4 * QP

NC = 2
NS = 16
NW = NC * NS
ROWS_PW = BATCH // NW
IDX_PW = ROWS_PW * HIST

CHUNK0 = 104
CHUNK1 = 96


def _transpose_kernel(i0, i1, i2, i3, o_ref):
    for q, iq in enumerate((i0, i1, i2, i3)):
        o_ref[:, 32 * q:32 * (q + 1)] = iq[...].T


LASTB = (VOCAB - 1) // TBW


@jax.jit
def _transpose(tT):
    return pl.pallas_call(
        _transpose_kernel,
        grid=(TNB,),
        in_specs=[pl.BlockSpec(
            (32, TBW),
            (lambda j, q=q: (0, jnp.minimum(TNB * q + j, LASTB))))
            for q in range(4)],
        out_specs=pl.BlockSpec((TBW, 128), lambda j: (j, 0)),
        out_shape=jax.ShapeDtypeStruct((QP, 128), jnp.float32),
    )(tT, tT, tT, tT)


def _pool_kernel(x_hbm, table_hbm, out_hbm, idx_v, rows_a, rows_b, pool_v,
                 sem_a, sem_b):
    wid = lax.axis_index("s") * NC + lax.axis_index("c")
    base = wid * ROWS_PW
    pltpu.sync_copy(x_hbm.at[pl.ds(base * HIST, IDX_PW)], idx_v)

    def remap_body(i, carry):
        for u in range(4):
            v = idx_v[pl.ds((i * 4 + u) * 16, 16)]
            w = v * 4
            for k in (1, 2, 3):
                w = w - jnp.where(v >= k * QP,
                                  jnp.int32(4 * QP - 1), jnp.int32(0))
            idx_v[pl.ds((i * 4 + u) * 16, 16)] = w
        return carry

    lax.fori_loop(0, IDX_PW // 64, remap_body, 0)

    inv = jnp.full((16,), 1.0 / HIST, dtype=jnp.float32)

    def issue(r, buf, sem):
        pltpu.async_copy(
            table_hbm.at[idx_v.at[pl.ds(r * HIST, CHUNK0)]],
            buf.at[pl.ds(0, CHUNK0)], sem)
        pltpu.async_copy(
            table_hbm.at[idx_v.at[pl.ds(r * HIST + CHUNK0, CHUNK1)]],
            buf.at[pl.ds(CHUNK0, CHUNK1)], sem)

    def drain(r, buf, sem):
        pltpu.make_async_copy(
            table_hbm.at[idx_v.at[pl.ds(r * HIST, CHUNK0)]],
            buf.at[pl.ds(0, CHUNK0)], sem).wait()
        pltpu.make_async_copy(
            table_hbm.at[idx_v.at[pl.ds(r * HIST + CHUNK0, CHUNK1)]],
            buf.at[pl.ds(CHUNK0, CHUNK1)], sem).wait()

    def accumulate(buf):
        zero = jnp.zeros((16,), dtype=jnp.float32)

        def acc_body(k, accs):
            a0, a1, a2, a3 = accs
            r4 = k * 4
            a0 = a0 + buf[r4, pl.ds(0, 16)]
            a1 = a1 + buf[r4, pl.ds(16, 16)]
            a2 = a2 + buf[r4 + 1, pl.ds(0, 16)]
            a3 = a3 + buf[r4 + 1, pl.ds(16, 16)]
            a0 = a0 + buf[r4 + 2, pl.ds(0, 16)]
            a1 = a1 + buf[r4 + 2, pl.ds(16, 16)]
            a2 = a2 + buf[r4 + 3, pl.ds(0, 16)]
            a3 = a3 + buf[r4 + 3, pl.ds(16, 16)]
            return (a0, a1, a2, a3)

        a0, a1, a2, a3 = lax.fori_loop(
            0, HIST // 4, acc_body, (zero, zero, zero, zero))
        return a0 + a2, a1 + a3

    issue(0, rows_a, sem_a)
    issue(1, rows_b, sem_b)

    def row_body(i, carry):
        r_a = i * 2
        r_b = i * 2 + 1

        drain(r_a, rows_a, sem_a)

        @pl.when(r_a + 2 < ROWS_PW)
        def _():
            issue(r_a + 2, rows_a, sem_a)

        s0, s1 = accumulate(rows_a)
        pool_v[pl.ds(r_a * EMB, 16)] = s0 * inv
        pool_v[pl.ds(r_a * EMB + 16, 16)] = s1 * inv

        drain(r_b, rows_b, sem_b)

        @pl.when(r_b + 2 < ROWS_PW)
        def _():
            issue(r_b + 2, rows_b, sem_b)

        s0, s1 = accumulate(rows_b)
        pool_v[pl.ds(r_b * EMB, 16)] = s0 * inv
        pool_v[pl.ds(r_b * EMB + 16, 16)] = s1 * inv
        return carry

    lax.fori_loop(0, ROWS_PW // 2, row_body, 0)
    pltpu.sync_copy(pool_v, out_hbm.at[pl.ds(base * EMB, ROWS_PW * EMB)])


@jax.jit
def _pool(x, table):
    mesh = plsc.VectorSubcoreMesh(core_axis_name="c", subcore_axis_name="s")
    return pl.kernel(
        _pool_kernel,
        mesh=mesh,
        compiler_params=pltpu.CompilerParams(use_tc_tiling_on_sc=False),
        out_type=jax.ShapeDtypeStruct((BATCH * EMB,), jnp.float32),
        scratch_types=[
            pltpu.VMEM((IDX_PW,), jnp.int32),
            pltpu.VMEM((HIST, EMB), jnp.float32),
            pltpu.VMEM((HIST, EMB), jnp.float32),
            pltpu.VMEM((ROWS_PW * EMB,), jnp.float32),
            pltpu.SemaphoreType.DMA,
            pltpu.SemaphoreType.DMA,
        ],
    )(x, table)


def _mm_kernel(p_ref, w_ref, b_ref, o_ref):
    o_ref[...] = lax.dot_general(
        p_ref[...], w_ref[...],
        (((1,), (1,)), ((), ())),
        preferred_element_type=jnp.float32,
    ) + b_ref[...]


@jax.jit
def _linear(pooled, W, b):
    return pl.pallas_call(
        _mm_kernel,
        out_shape=jax.ShapeDtypeStruct((BATCH, NCLASS), jnp.float32),
    )(pooled, W, b.reshape(1, NCLASS))


def kernel(x, table, W, b):
    z = _transpose(table.T)
    pooled = _pool(x.reshape(BATCH * HIST),
                   z.reshape(VPAD, EMB)).reshape(BATCH, EMB)
    return _linear(pooled, W, b)


# speedup vs baseline: 6.7131x; 1.7666x over previous
import functools

import jax
import jax.numpy as jnp
from jax import lax
from jax.experimental import pallas as pl
from jax.experimental.pallas import tpu as pltpu
from jax.experimental.pallas import tpu_sc as plsc

VOCAB = 1000000
EMB = 32
NCLASS = 100
BATCH = 4096
HIST = 200

TBW = 4096
TNB = 62
QP = TBW * TNB
VPAD = 4 * QP

NC = 2
NS = 16
NW = NC * NS
ROWS_PW = BATCH // NW
IDX_PW = ROWS_PW * HIST

CHUNK0 = 104
CHUNK1 = 96


def _transpose_kernel(i0, i1, i2, i3, o_ref):
    o_ref[...] = jnp.concatenate(
        [iq[...] for iq in (i0, i1, i2, i3)], axis=0).T


LASTB = (VOCAB - 1) // TBW


@jax.jit
def _transpose(tT):
    return pl.pallas_call(
        _transpose_kernel,
        grid=(TNB,),
        in_specs=[pl.BlockSpec(
            (32, TBW),
            (lambda j, q=q: (0, jnp.minimum(TNB * q + j, LASTB))))
            for q in range(4)],
        out_specs=pl.BlockSpec((TBW, 128), lambda j: (j, 0)),
        out_shape=jax.ShapeDtypeStruct((QP, 128), jnp.float32),
    )(tT, tT, tT, tT)


def _pool_kernel(x_hbm, table_hbm, out_hbm, idx_v, rows_a, rows_b, pool_v,
                 sem_a, sem_b):
    wid = lax.axis_index("s") * NC + lax.axis_index("c")
    base = wid * ROWS_PW
    pltpu.sync_copy(x_hbm.at[pl.ds(base * HIST, IDX_PW)], idx_v)

    def remap_body(i, carry):
        for u in range(4):
            v = idx_v[pl.ds((i * 4 + u) * 16, 16)]
            w = v * 4
            for k in (1, 2, 3):
                w = w - jnp.where(v >= k * QP,
                                  jnp.int32(4 * QP - 1), jnp.int32(0))
            idx_v[pl.ds((i * 4 + u) * 16, 16)] = w
        return carry

    lax.fori_loop(0, IDX_PW // 64, remap_body, 0)

    inv = jnp.full((16,), 1.0 / HIST, dtype=jnp.float32)

    def issue(r, buf, sem):
        pltpu.async_copy(
            table_hbm.at[idx_v.at[pl.ds(r * HIST, CHUNK0)]],
            buf.at[pl.ds(0, CHUNK0)], sem)
        pltpu.async_copy(
            table_hbm.at[idx_v.at[pl.ds(r * HIST + CHUNK0, CHUNK1)]],
            buf.at[pl.ds(CHUNK0, CHUNK1)], sem)

    def drain(r, buf, sem):
        pltpu.make_async_copy(
            table_hbm.at[idx_v.at[pl.ds(r * HIST, CHUNK0)]],
            buf.at[pl.ds(0, CHUNK0)], sem).wait()
        pltpu.make_async_copy(
            table_hbm.at[idx_v.at[pl.ds(r * HIST + CHUNK0, CHUNK1)]],
            buf.at[pl.ds(CHUNK0, CHUNK1)], sem).wait()

    def accumulate(buf):
        zero = jnp.zeros((16,), dtype=jnp.float32)

        def acc_body(k, accs):
            a0, a1, a2, a3 = accs
            r4 = k * 4
            a0 = a0 + buf[r4, pl.ds(0, 16)]
            a1 = a1 + buf[r4, pl.ds(16, 16)]
            a2 = a2 + buf[r4 + 1, pl.ds(0, 16)]
            a3 = a3 + buf[r4 + 1, pl.ds(16, 16)]
            a0 = a0 + buf[r4 + 2, pl.ds(0, 16)]
            a1 = a1 + buf[r4 + 2, pl.ds(16, 16)]
            a2 = a2 + buf[r4 + 3, pl.ds(0, 16)]
            a3 = a3 + buf[r4 + 3, pl.ds(16, 16)]
            return (a0, a1, a2, a3)

        a0, a1, a2, a3 = lax.fori_loop(
            0, HIST // 4, acc_body, (zero, zero, zero, zero))
        return a0 + a2, a1 + a3

    issue(0, rows_a, sem_a)
    issue(1, rows_b, sem_b)

    def row_body(i, carry):
        r_a = i * 2
        r_b = i * 2 + 1

        drain(r_a, rows_a, sem_a)

        @pl.when(r_a + 2 < ROWS_PW)
        def _():
            issue(r_a + 2, rows_a, sem_a)

        s0, s1 = accumulate(rows_a)
        pool_v[pl.ds(r_a * EMB, 16)] = s0 * inv
        pool_v[pl.ds(r_a * EMB + 16, 16)] = s1 * inv

        drain(r_b, rows_b, sem_b)

        @pl.when(r_b + 2 < ROWS_PW)
        def _():
            issue(r_b + 2, rows_b, sem_b)

        s0, s1 = accumulate(rows_b)
        pool_v[pl.ds(r_b * EMB, 16)] = s0 * inv
        pool_v[pl.ds(r_b * EMB + 16, 16)] = s1 * inv
        return carry

    lax.fori_loop(0, ROWS_PW // 2, row_body, 0)
    pltpu.sync_copy(pool_v, out_hbm.at[pl.ds(base * EMB, ROWS_PW * EMB)])


@jax.jit
def _pool(x, table):
    mesh = plsc.VectorSubcoreMesh(core_axis_name="c", subcore_axis_name="s")
    return pl.kernel(
        _pool_kernel,
        mesh=mesh,
        compiler_params=pltpu.CompilerParams(use_tc_tiling_on_sc=False),
        out_type=jax.ShapeDtypeStruct((BATCH * EMB,), jnp.float32),
        scratch_types=[
            pltpu.VMEM((IDX_PW,), jnp.int32),
            pltpu.VMEM((HIST, EMB), jnp.float32),
            pltpu.VMEM((HIST, EMB), jnp.float32),
            pltpu.VMEM((ROWS_PW * EMB,), jnp.float32),
            pltpu.SemaphoreType.DMA,
            pltpu.SemaphoreType.DMA,
        ],
    )(x, table)


def _mm_kernel(p_ref, w_ref, b_ref, o_ref):
    o_ref[...] = lax.dot_general(
        p_ref[...], w_ref[...],
        (((1,), (1,)), ((), ())),
        preferred_element_type=jnp.float32,
    ) + b_ref[...]


@jax.jit
def _linear(pooled, W, b):
    return pl.pallas_call(
        _mm_kernel,
        out_shape=jax.ShapeDtypeStruct((BATCH, NCLASS), jnp.float32),
    )(pooled, W, b.reshape(1, NCLASS))


def kernel(x, table, W, b):
    z = _transpose(table.T)
    pooled = _pool(x.reshape(BATCH * HIST),
                   z.reshape(VPAD, EMB)).reshape(BATCH, EMB)
    return _linear(pooled, W, b)


# speedup vs baseline: 8.3176x; 1.2390x over previous
import functools

import jax
import jax.numpy as jnp
from jax import lax
from jax.experimental import pallas as pl
from jax.experimental.pallas import tpu as pltpu
from jax.experimental.pallas import tpu_sc as plsc

VOCAB = 1000000
EMB = 32
NCLASS = 100
BATCH = 4096
HIST = 200

TBW = 4096
TNB = 31
QP = TBW * TNB
NSPLIT = 8
VPAD = NSPLIT * QP
LASTB = (VOCAB - 1) // TBW

NC = 2
NS = 16
NW = NC * NS
ROWS_PW = BATCH // NW
IDX_PW = ROWS_PW * HIST

CHUNK0 = 104
CHUNK1 = 96


def _transpose_kernel(*refs):
    o_ref = refs[-1]
    packed = []
    for iq in refs[:-1]:
        t = iq[...]
        lo = t[0:16, :].astype(jnp.bfloat16)
        hi = t[16:32, :].astype(jnp.bfloat16)
        lo_u = lax.bitcast_convert_type(lo, jnp.uint16).astype(jnp.uint32)
        hi_u = lax.bitcast_convert_type(hi, jnp.uint16).astype(jnp.uint32)
        packed.append(
            lax.bitcast_convert_type(lo_u | (hi_u << 16), jnp.float32))
    o_ref[...] = jnp.concatenate(packed, axis=0).T


@jax.jit
def _transpose(tT):
    return pl.pallas_call(
        _transpose_kernel,
        grid=(TNB,),
        in_specs=[pl.BlockSpec(
            (32, TBW),
            (lambda j, q=q: (0, jnp.minimum(TNB * q + j, LASTB))))
            for q in range(NSPLIT)],
        out_specs=pl.BlockSpec((TBW, 128), lambda j: (j, 0)),
        out_shape=jax.ShapeDtypeStruct((QP, 128), jnp.float32),
    )(*([tT] * NSPLIT))


def _pool_kernel(x_hbm, table_hbm, out_hbm, idx_v, rows_a, rows_b, pool_v,
                 sem_a, sem_b):
    wid = lax.axis_index("s") * NC + lax.axis_index("c")
    base = wid * ROWS_PW
    pltpu.sync_copy(x_hbm.at[pl.ds(base * HIST, IDX_PW)], idx_v)

    def remap_body(i, carry):
        for u in range(4):
            v = idx_v[pl.ds((i * 4 + u) * 16, 16)]
            q = ((v >> 12) * 529) >> 14
            idx_v[pl.ds((i * 4 + u) * 16, 16)] = (
                (v << 3) - q * (NSPLIT * QP - 1))
        return carry

    lax.fori_loop(0, IDX_PW // 64, remap_body, 0)

    inv = jnp.full((16,), 1.0 / HIST, dtype=jnp.float32)

    def issue(r, buf, sem):
        pltpu.async_copy(
            table_hbm.at[idx_v.at[pl.ds(r * HIST, CHUNK0)]],
            buf.at[pl.ds(0, CHUNK0)], sem)
        pltpu.async_copy(
            table_hbm.at[idx_v.at[pl.ds(r * HIST + CHUNK0, CHUNK1)]],
            buf.at[pl.ds(CHUNK0, CHUNK1)], sem)

    def drain(r, buf, sem):
        pltpu.make_async_copy(
            table_hbm.at[idx_v.at[pl.ds(r * HIST, CHUNK0)]],
            buf.at[pl.ds(0, CHUNK0)], sem).wait()
        pltpu.make_async_copy(
            table_hbm.at[idx_v.at[pl.ds(r * HIST + CHUNK0, CHUNK1)]],
            buf.at[pl.ds(CHUNK0, CHUNK1)], sem).wait()

    mask_hi = jnp.full((16,), -65536, dtype=jnp.int32)

    def load2(buf, r):
        p = lax.bitcast_convert_type(buf[r, pl.ds(0, 16)], jnp.int32)
        lo = lax.bitcast_convert_type(p << 16, jnp.float32)
        hi = lax.bitcast_convert_type(p & mask_hi, jnp.float32)
        return lo, hi

    def accumulate(buf):
        zero = jnp.zeros((16,), dtype=jnp.float32)

        def acc_body(k, accs):
            a0, a1, a2, a3 = accs
            r4 = k * 4
            lo, hi = load2(buf, r4)
            a0 = a0 + lo
            a1 = a1 + hi
            lo, hi = load2(buf, r4 + 1)
            a2 = a2 + lo
            a3 = a3 + hi
            lo, hi = load2(buf, r4 + 2)
            a0 = a0 + lo
            a1 = a1 + hi
            lo, hi = load2(buf, r4 + 3)
            a2 = a2 + lo
            a3 = a3 + hi
            return (a0, a1, a2, a3)

        a0, a1, a2, a3 = lax.fori_loop(
            0, HIST // 4, acc_body, (zero, zero, zero, zero))
        return a0 + a2, a1 + a3

    issue(0, rows_a, sem_a)
    issue(1, rows_b, sem_b)

    def row_body(i, carry):
        r_a = i * 2
        r_b = i * 2 + 1

        drain(r_a, rows_a, sem_a)

        @pl.when(r_a + 2 < ROWS_PW)
        def _():
            issue(r_a + 2, rows_a, sem_a)

        s0, s1 = accumulate(rows_a)
        pool_v[pl.ds(r_a * EMB, 16)] = s0 * inv
        pool_v[pl.ds(r_a * EMB + 16, 16)] = s1 * inv

        drain(r_b, rows_b, sem_b)

        @pl.when(r_b + 2 < ROWS_PW)
        def _():
            issue(r_b + 2, rows_b, sem_b)

        s0, s1 = accumulate(rows_b)
        pool_v[pl.ds(r_b * EMB, 16)] = s0 * inv
        pool_v[pl.ds(r_b * EMB + 16, 16)] = s1 * inv
        return carry

    lax.fori_loop(0, ROWS_PW // 2, row_body, 0)
    pltpu.sync_copy(pool_v, out_hbm.at[pl.ds(base * EMB, ROWS_PW * EMB)])


@jax.jit
def _pool(x, table):
    mesh = plsc.VectorSubcoreMesh(core_axis_name="c", subcore_axis_name="s")
    return pl.kernel(
        _pool_kernel,
        mesh=mesh,
        compiler_params=pltpu.CompilerParams(use_tc_tiling_on_sc=False),
        out_type=jax.ShapeDtypeStruct((BATCH * EMB,), jnp.float32),
        scratch_types=[
            pltpu.VMEM((IDX_PW,), jnp.int32),
            pltpu.VMEM((HIST, 16), jnp.float32),
            pltpu.VMEM((HIST, 16), jnp.float32),
            pltpu.VMEM((ROWS_PW * EMB,), jnp.float32),
            pltpu.SemaphoreType.DMA,
            pltpu.SemaphoreType.DMA,
        ],
    )(x, table)


def _mm_kernel(p_ref, w_ref, b_ref, o_ref):
    o_ref[...] = lax.dot_general(
        p_ref[...], w_ref[...],
        (((1,), (1,)), ((), ())),
        preferred_element_type=jnp.float32,
    ) + b_ref[...]


@jax.jit
def _linear(pooled, W, b):
    return pl.pallas_call(
        _mm_kernel,
        out_shape=jax.ShapeDtypeStruct((BATCH, NCLASS), jnp.float32),
    )(pooled, W, b.reshape(1, NCLASS))


def kernel(x, table, W, b):
    z = _transpose(table.T)
    pooled = _pool(x.reshape(BATCH * HIST),
                   z.reshape(VPAD, 16)).reshape(BATCH, EMB)
    return _linear(pooled, W, b)


# speedup vs baseline: 8.5229x; 1.0247x over previous
import functools

import jax
import jax.numpy as jnp
from jax import lax
from jax.experimental import pallas as pl
from jax.experimental.pallas import tpu as pltpu
from jax.experimental.pallas import tpu_sc as plsc

VOCAB = 1000000
EMB = 32
NCLASS = 100
BATCH = 4096
HIST = 200

TBW = 8192
TNB = 16
QP = TBW * TNB
NSPLIT = 8
VPAD = NSPLIT * QP
LASTB = (VOCAB - 1) // TBW

NC = 2
NS = 16
NW = NC * NS
ROWS_PW = BATCH // NW
IDX_PW = ROWS_PW * HIST

CHUNK0 = 104
CHUNK1 = 96


def _transpose_kernel(*refs):
    o_ref = refs[-1]
    packed = []
    for iq in refs[:-1]:
        t = iq[...]
        lo = t[0:16, :].astype(jnp.bfloat16)
        hi = t[16:32, :].astype(jnp.bfloat16)
        lo_u = lax.bitcast_convert_type(lo, jnp.uint16).astype(jnp.uint32)
        hi_u = lax.bitcast_convert_type(hi, jnp.uint16).astype(jnp.uint32)
        packed.append(
            lax.bitcast_convert_type(lo_u | (hi_u << 16), jnp.float32))
    o_ref[...] = jnp.concatenate(packed, axis=0).T


@jax.jit
def _transpose(tT):
    return pl.pallas_call(
        _transpose_kernel,
        grid=(TNB,),
        in_specs=[pl.BlockSpec(
            (32, TBW),
            (lambda j, q=q: (0, jnp.minimum(TNB * q + j, LASTB))))
            for q in range(NSPLIT)],
        out_specs=pl.BlockSpec((TBW, 128), lambda j: (j, 0)),
        out_shape=jax.ShapeDtypeStruct((QP, 128), jnp.float32),
    )(*([tT] * NSPLIT))


def _pool_kernel(x_hbm, table_hbm, out_hbm, idx_v, rows_a, rows_b, pool_v,
                 sem_a, sem_b):
    wid = lax.axis_index("s") * NC + lax.axis_index("c")
    base = wid * ROWS_PW
    pltpu.sync_copy(x_hbm.at[pl.ds(base * HIST, IDX_PW)], idx_v)

    def remap_body(i, carry):
        for u in range(4):
            v = idx_v[pl.ds((i * 4 + u) * 16, 16)]
            q = v >> 17
            idx_v[pl.ds((i * 4 + u) * 16, 16)] = (v << 3) - q * (VPAD - 1)
        return carry

    lax.fori_loop(0, IDX_PW // 64, remap_body, 0)

    inv = jnp.full((16,), 1.0 / HIST, dtype=jnp.float32)

    def issue(r, buf, sem):
        pltpu.async_copy(
            table_hbm.at[idx_v.at[pl.ds(r * HIST, CHUNK0)]],
            buf.at[pl.ds(0, CHUNK0)], sem)
        pltpu.async_copy(
            table_hbm.at[idx_v.at[pl.ds(r * HIST + CHUNK0, CHUNK1)]],
            buf.at[pl.ds(CHUNK0, CHUNK1)], sem)

    def drain(r, buf, sem):
        pltpu.make_async_copy(
            table_hbm.at[idx_v.at[pl.ds(r * HIST, CHUNK0)]],
            buf.at[pl.ds(0, CHUNK0)], sem).wait()
        pltpu.make_async_copy(
            table_hbm.at[idx_v.at[pl.ds(r * HIST + CHUNK0, CHUNK1)]],
            buf.at[pl.ds(CHUNK0, CHUNK1)], sem).wait()

    mask_hi = jnp.full((16,), -65536, dtype=jnp.int32)

    def load2(buf, r):
        p = lax.bitcast_convert_type(buf[r, pl.ds(0, 16)], jnp.int32)
        lo = lax.bitcast_convert_type(p << 16, jnp.float32)
        hi = lax.bitcast_convert_type(p & mask_hi, jnp.float32)
        return lo, hi

    def accumulate(buf):
        zero = jnp.zeros((16,), dtype=jnp.float32)

        def acc_body(k, accs):
            a0, a1, a2, a3 = accs
            r8 = k * 8
            for u in range(8):
                lo, hi = load2(buf, r8 + u)
                if u % 2 == 0:
                    a0 = a0 + lo
                    a1 = a1 + hi
                else:
                    a2 = a2 + lo
                    a3 = a3 + hi
            return (a0, a1, a2, a3)

        a0, a1, a2, a3 = lax.fori_loop(
            0, HIST // 8, acc_body, (zero, zero, zero, zero))
        return a0 + a2, a1 + a3

    issue(0, rows_a, sem_a)
    issue(1, rows_b, sem_b)

    def row_body(i, carry):
        r_a = i * 2
        r_b = i * 2 + 1

        drain(r_a, rows_a, sem_a)

        @pl.when(r_a + 2 < ROWS_PW)
        def _():
            issue(r_a + 2, rows_a, sem_a)

        s0, s1 = accumulate(rows_a)
        pool_v[pl.ds(r_a * EMB, 16)] = s0 * inv
        pool_v[pl.ds(r_a * EMB + 16, 16)] = s1 * inv

        drain(r_b, rows_b, sem_b)

        @pl.when(r_b + 2 < ROWS_PW)
        def _():
            issue(r_b + 2, rows_b, sem_b)

        s0, s1 = accumulate(rows_b)
        pool_v[pl.ds(r_b * EMB, 16)] = s0 * inv
        pool_v[pl.ds(r_b * EMB + 16, 16)] = s1 * inv
        return carry

    lax.fori_loop(0, ROWS_PW // 2, row_body, 0)
    pltpu.sync_copy(pool_v, out_hbm.at[pl.ds(base * EMB, ROWS_PW * EMB)])


@jax.jit
def _pool(x, table):
    mesh = plsc.VectorSubcoreMesh(core_axis_name="c", subcore_axis_name="s")
    return pl.kernel(
        _pool_kernel,
        mesh=mesh,
        compiler_params=pltpu.CompilerParams(use_tc_tiling_on_sc=False),
        out_type=jax.ShapeDtypeStruct((BATCH * EMB,), jnp.float32),
        scratch_types=[
            pltpu.VMEM((IDX_PW,), jnp.int32),
            pltpu.VMEM((HIST, 16), jnp.float32),
            pltpu.VMEM((HIST, 16), jnp.float32),
            pltpu.VMEM((ROWS_PW * EMB,), jnp.float32),
            pltpu.SemaphoreType.DMA,
            pltpu.SemaphoreType.DMA,
        ],
    )(x, table)


def _mm_kernel(p_ref, w_ref, b_ref, o_ref):
    o_ref[...] = lax.dot_general(
        p_ref[...], w_ref[...],
        (((1,), (1,)), ((), ())),
        preferred_element_type=jnp.float32,
    ) + b_ref[...]


@jax.jit
def _linear(pooled, W, b):
    return pl.pallas_call(
        _mm_kernel,
        out_shape=jax.ShapeDtypeStruct((BATCH, NCLASS), jnp.float32),
    )(pooled, W, b.reshape(1, NCLASS))


def kernel(x, table, W, b):
    z = _transpose(table.T)
    pooled = _pool(x.reshape(BATCH * HIST),
                   z.reshape(VPAD, 16)).reshape(BATCH, EMB)
    return _linear(pooled, W, b)
